# Initial kernel scaffold; baseline (speedup 1.0000x reference)
import jax
import jax.numpy as jnp
from jax.experimental import pallas as pl


def kernel(cls_heads, reg_heads, batch_anchors):
    raise NotImplementedError("write your pallas kernel here")


# speedup vs baseline: 1.0554x; 1.0554x over previous
import jax
import jax.numpy as jnp
from jax import lax
from jax.experimental import pallas as pl
from jax.experimental.pallas import tpu as pltpu

_IMAGE_W = 1024
_IMAGE_H = 1024
_TOP_N = 1000
_MIN_SCORE = 0.05
_NMS_THR = 0.5
_MAX_DET = 100
_NEG = float("-inf")
_ONE_BITS = 0x3F800000

_INTERPRET = False


def _maxarg_body(x_ref, s_ref, c_ref):
    x = x_ref[0]
    n, c = x.shape
    mx = jnp.max(x, axis=1)
    ids = lax.broadcasted_iota(jnp.int32, (n, c), 1)
    am = jnp.min(jnp.where(x == mx[:, None], ids, c), axis=1)
    s_ref[0, 0, :] = mx
    c_ref[0, 0, :] = am


def _thresh_body(s_ref, thr_ref, cut_ref):
    bits = lax.bitcast_convert_type(s_ref[...], jnp.int32)
    r, n = bits.shape
    k = _TOP_N
    ids = lax.broadcasted_iota(jnp.int32, (r, n), 1)

    def vstep(_, lohi):
        lo, hi = lohi
        mid = lax.div(lo + hi, 2)
        cnt = jnp.sum((bits >= mid).astype(jnp.int32), axis=1, keepdims=True)
        ge = cnt >= k
        return jnp.where(ge, mid, lo), jnp.where(ge, hi, mid)

    lo0 = jnp.zeros((r, 1), jnp.int32)
    hi0 = jnp.full((r, 1), _ONE_BITS, jnp.int32)
    vk, _ = lax.fori_loop(0, 31, vstep, (lo0, hi0))

    g = jnp.sum((bits > vk).astype(jnp.int32), axis=1, keepdims=True)
    m = k - g
    tie = bits == vk

    def istep(_, lohi):
        lo2, hi2 = lohi
        mid = lax.div(lo2 + hi2, 2)
        cnt = jnp.sum((tie & (ids < mid)).astype(jnp.int32), axis=1,
                      keepdims=True)
        ge = cnt >= m
        return jnp.where(ge, lo2, mid), jnp.where(ge, mid, hi2)

    lo20 = jnp.zeros((r, 1), jnp.int32)
    hi20 = jnp.full((r, 1), n, jnp.int32)
    _, cut = lax.fori_loop(0, 15, istep, (lo20, hi20))

    thr_ref[...] = vk
    cut_ref[...] = cut


def _decode_body(s_ref, c_ref, thr_ref, cut_ref,
                 rx_ref, ry_ref, rw_ref, rh_ref,
                 ax1_ref, ay1_ref, ax2_ref, ay2_ref,
                 se_ref, cf_ref, x1_ref, y1_ref, x2_ref, y2_ref, ar_ref):
    s = s_ref[0]
    n = s.shape[1]
    bits = lax.bitcast_convert_type(s, jnp.int32)
    ids = lax.broadcasted_iota(jnp.int32, (1, n), 1)
    thr = thr_ref[0, 0, 0]
    cut = cut_ref[0, 0, 0]
    sel = (bits > thr) | ((bits == thr) & (ids < cut))
    valid = sel & (s > _MIN_SCORE)

    ax1 = ax1_ref[0]
    ay1 = ay1_ref[0]
    aw = ax2_ref[0] - ax1
    ah = ay2_ref[0] - ay1
    acx = ax1 + 0.5 * aw
    acy = ay1 + 0.5 * ah
    w = jnp.exp(rw_ref[0] * 0.2) * aw
    h = jnp.exp(rh_ref[0] * 0.2) * ah
    cx = rx_ref[0] * 0.1 * aw + acx
    cy = ry_ref[0] * 0.1 * ah + acy
    bx1 = (cx - 0.5 * w).astype(jnp.int32)
    by1 = (cy - 0.5 * h).astype(jnp.int32)
    bx2 = (cx + 0.5 * w).astype(jnp.int32)
    by2 = (cy + 0.5 * h).astype(jnp.int32)
    x1 = jnp.maximum(bx1, 0).astype(jnp.float32)
    y1 = jnp.maximum(by1, 0).astype(jnp.float32)
    x2 = jnp.minimum(bx2, _IMAGE_W - 1).astype(jnp.float32)
    y2 = jnp.minimum(by2, _IMAGE_H - 1).astype(jnp.float32)

    se_ref[0] = jnp.where(valid, s, _NEG)
    cf_ref[0] = c_ref[0].astype(jnp.float32)
    x1_ref[0] = x1
    y1_ref[0] = y1
    x2_ref[0] = x2
    y2_ref[0] = y2
    ar_ref[0] = (x2 - x1) * (y2 - y1)


def _nms_body(s_ref, c_ref, x1_ref, y1_ref, x2_ref, y2_ref, ar_ref,
              os_ref, oc_ref, ob0_ref, ob1_ref, ob2_ref, ob3_ref):
    s0 = s_ref[...]
    b, mM = s0.shape
    cls = c_ref[...]
    x1 = x1_ref[...]
    y1 = y1_ref[...]
    x2 = x2_ref[...]
    y2 = y2_ref[...]
    ar = ar_ref[...]
    ids = lax.broadcasted_iota(jnp.int32, (b, mM), 1)
    jj = lax.broadcasted_iota(jnp.int32, (b, 128), 1)
    zacc = jnp.full((b, 128), -1.0, jnp.float32)

    def step(j, carry):
        scur, aos, aoc, a0, a1, a2, a3 = carry
        m = jnp.max(scur, axis=1, keepdims=True)
        has = m > _NEG
        pos = jnp.min(jnp.where(scur == m, ids, mM), axis=1, keepdims=True)
        sel = ids == pos
        bx1 = jnp.sum(jnp.where(sel, x1, 0.0), axis=1, keepdims=True)
        by1 = jnp.sum(jnp.where(sel, y1, 0.0), axis=1, keepdims=True)
        bx2 = jnp.sum(jnp.where(sel, x2, 0.0), axis=1, keepdims=True)
        by2 = jnp.sum(jnp.where(sel, y2, 0.0), axis=1, keepdims=True)
        bar = jnp.sum(jnp.where(sel, ar, 0.0), axis=1, keepdims=True)
        bcl = jnp.sum(jnp.where(sel, cls, 0.0), axis=1, keepdims=True)
        xx1 = jnp.maximum(x1, bx1)
        yy1 = jnp.maximum(y1, by1)
        xx2 = jnp.minimum(x2, bx2)
        yy2 = jnp.minimum(y2, by2)
        inter = jnp.maximum(xx2 - xx1, 0.0) * jnp.maximum(yy2 - yy1, 0.0)
        iou = inter / (ar + bar - inter)
        kill = (iou > _NMS_THR) | sel
        snew = jnp.where(kill, _NEG, scur)
        scur = jnp.where(has, snew, scur)
        hit = jj == j
        aos = jnp.where(hit & has, m, aos)
        aoc = jnp.where(hit & has, bcl, aoc)
        a0 = jnp.where(hit & has, bx1, a0)
        a1 = jnp.where(hit & has, by1, a1)
        a2 = jnp.where(hit & has, bx2, a2)
        a3 = jnp.where(hit & has, by2, a3)
        return scur, aos, aoc, a0, a1, a2, a3

    init = (s0, zacc, zacc, zacc, zacc, zacc, zacc)
    _, aos, aoc, a0, a1, a2, a3 = lax.fori_loop(0, _MAX_DET, step, init)
    os_ref[...] = aos[:, :_MAX_DET]
    oc_ref[...] = aoc[:, :_MAX_DET]
    ob0_ref[...] = a0[:, :_MAX_DET]
    ob1_ref[...] = a1[:, :_MAX_DET]
    ob2_ref[...] = a2[:, :_MAX_DET]
    ob3_ref[...] = a3[:, :_MAX_DET]


def kernel(cls_heads, reg_heads, batch_anchors):
    L, B, N, C = cls_heads.shape
    R = L * B

    row3_spec = pl.BlockSpec((1, 1, N), lambda i: (i, 0, 0))
    scores3, classes3 = pl.pallas_call(
        _maxarg_body,
        grid=(R,),
        in_specs=[pl.BlockSpec((1, N, C), lambda i: (i, 0, 0))],
        out_specs=[row3_spec, row3_spec],
        out_shape=[jax.ShapeDtypeStruct((R, 1, N), jnp.float32),
                   jax.ShapeDtypeStruct((R, 1, N), jnp.int32)],
        interpret=_INTERPRET,
    )(cls_heads.reshape(R, N, C))
    scores = scores3.reshape(R, N)
    classes = classes3.reshape(R, N)

    thr, cut = pl.pallas_call(
        _thresh_body,
        out_shape=[jax.ShapeDtypeStruct((R, 1), jnp.int32),
                   jax.ShapeDtypeStruct((R, 1), jnp.int32)],
        interpret=_INTERPRET,
    )(scores)

    def to_bl(a):
        return a.reshape(L, B, N).transpose(1, 0, 2).reshape(R, N)

    s_bl = to_bl(scores)
    c_bl = to_bl(classes)
    thr_bl = thr.reshape(L, B, 1).transpose(1, 0, 2).reshape(R, 1)
    cut_bl = cut.reshape(L, B, 1).transpose(1, 0, 2).reshape(R, 1)
    reg_t = reg_heads.transpose(1, 0, 2, 3).reshape(R, N, 4)
    anc_t = batch_anchors.transpose(1, 0, 2, 3).reshape(R, N, 4)
    regc = [reg_t[:, :, i] for i in range(4)]
    ancc = [anc_t[:, :, i] for i in range(4)]

    one_spec = pl.BlockSpec((1, 1, 1), lambda i: (i, 0, 0))
    r3 = lambda a: a.reshape(R, 1, N)
    outs = pl.pallas_call(
        _decode_body,
        grid=(R,),
        in_specs=[row3_spec, row3_spec, one_spec, one_spec] + [row3_spec] * 8,
        out_specs=[row3_spec] * 7,
        out_shape=[jax.ShapeDtypeStruct((R, 1, N), jnp.float32)] * 7,
        interpret=_INTERPRET,
    )(r3(s_bl), r3(c_bl), thr_bl.reshape(R, 1, 1), cut_bl.reshape(R, 1, 1),
      *[r3(a) for a in regc], *[r3(a) for a in ancc])
    se, cf, x1, y1, x2, y2, ar = [o.reshape(B, L * N) for o in outs]

    os_, oc_, ob0, ob1, ob2, ob3 = pl.pallas_call(
        _nms_body,
        out_shape=[jax.ShapeDtypeStruct((B, _MAX_DET), jnp.float32)] * 6,
        interpret=_INTERPRET,
    )(se, cf, x1, y1, x2, y2, ar)

    boxes = jnp.stack([ob0, ob1, ob2, ob3], axis=-1)
    return os_, oc_, boxes
